# Initial kernel scaffold; baseline (speedup 1.0000x reference)
import jax
import jax.numpy as jnp
from jax.experimental import pallas as pl


def kernel(features, labels, centers):
    raise NotImplementedError("write your pallas kernel here")


# speedup vs baseline: 1.0341x; 1.0341x over previous
import functools

import jax
import jax.numpy as jnp
from jax import lax
from jax.experimental import pallas as pl
from jax.experimental.pallas import tpu as pltpu
from jax.experimental.pallas import tpu_sc as plsc

_LANES = 16
_NUM_CORES = 2
_NUM_SUBCORES = 16
_NW = _NUM_CORES * _NUM_SUBCORES


def _make_sc_kernel(batch, feat_dim):
    rows_per_w = batch // _NW
    chunk = 128
    nchunk = rows_per_w // chunk
    vecs_per_row = feat_dim // _LANES

    mesh = plsc.VectorSubcoreMesh(core_axis_name="c", subcore_axis_name="s")

    @functools.partial(
        pl.kernel,
        out_type=jax.ShapeDtypeStruct((_NW * _LANES,), jnp.float32),
        mesh=mesh,
        scratch_types=[
            pltpu.VMEM((chunk,), jnp.int32),
            pltpu.VMEM((chunk, feat_dim), jnp.float32),
            pltpu.VMEM((chunk, feat_dim), jnp.float32),
            pltpu.VMEM((_LANES,), jnp.float32),
            pltpu.SemaphoreType.DMA,
        ],
    )
    def sc_kernel(feat_hbm, labels_hbm, centers_hbm, out_hbm,
                  idx_v, cent_v, feat_v, acc_v, sem):
        wid = lax.axis_index("s") * _NUM_CORES + lax.axis_index("c")
        base = wid * rows_per_w

        accs = [jnp.zeros((_LANES,), jnp.float32) for _ in range(vecs_per_row)]
        for ch in range(nchunk):
            start = base + ch * chunk
            pltpu.sync_copy(labels_hbm.at[pl.ds(start, chunk)], idx_v)
            gather = pltpu.async_copy(centers_hbm.at[idx_v], cent_v, sem)
            pltpu.sync_copy(feat_hbm.at[pl.ds(start, chunk)], feat_v)
            gather.wait()

            def row_body(r, accs):
                out = []
                for j in range(vecs_per_row):
                    f = feat_v[r, pl.ds(j * _LANES, _LANES)]
                    c = cent_v[r, pl.ds(j * _LANES, _LANES)]
                    d = f - c
                    out.append(accs[j] + d * d)
                return tuple(out)

            accs = lax.fori_loop(0, chunk, row_body, tuple(accs))

        total = accs[0]
        for j in range(1, vecs_per_row):
            total = total + accs[j]
        acc_v[...] = total
        pltpu.sync_copy(acc_v, out_hbm.at[pl.ds(wid * _LANES, _LANES)])

    return sc_kernel


def kernel(features, labels, centers):
    batch, feat_dim = features.shape
    sc = _make_sc_kernel(batch, feat_dim)
    partials = sc(features, labels.astype(jnp.int32), centers)
    return jnp.sum(partials) / jnp.float32(batch)


# speedup vs baseline: 1.1779x; 1.1390x over previous
import functools

import jax
import jax.numpy as jnp
from jax import lax
from jax.experimental import pallas as pl
from jax.experimental.pallas import tpu as pltpu
from jax.experimental.pallas import tpu_sc as plsc

_LANES = 16
_NUM_CORES = 2
_NUM_SUBCORES = 16
_NW = _NUM_CORES * _NUM_SUBCORES


def _make_sc_kernel(batch, feat_dim):
    rows_per_w = batch // _NW
    chunk = 128
    nchunk = rows_per_w // chunk
    vecs_per_row = feat_dim // _LANES

    mesh = plsc.VectorSubcoreMesh(core_axis_name="c", subcore_axis_name="s")

    @functools.partial(
        pl.kernel,
        out_type=jax.ShapeDtypeStruct((_NW * _LANES,), jnp.float32),
        mesh=mesh,
        scratch_types=[
            pltpu.VMEM((chunk,), jnp.int32),
            pltpu.VMEM((chunk,), jnp.int32),
            pltpu.VMEM((chunk, feat_dim), jnp.float32),
            pltpu.VMEM((chunk, feat_dim), jnp.float32),
            pltpu.VMEM((rows_per_w, feat_dim), jnp.float32),
            pltpu.VMEM((_LANES,), jnp.float32),
            pltpu.SemaphoreType.DMA,
            pltpu.SemaphoreType.DMA,
            pltpu.SemaphoreType.DMA,
        ],
    )
    def sc_kernel(feat_hbm, labels_hbm, centers_hbm, out_hbm,
                  idx0, idx1, cent0, cent1, feat_v, acc_v,
                  sem_f, sem_g0, sem_g1):
        wid = lax.axis_index("s") * _NUM_CORES + lax.axis_index("c")
        base = wid * rows_per_w

        idxs, cents, sems = [idx0, idx1], [cent0, cent1], [sem_g0, sem_g1]
        fcopy = pltpu.async_copy(feat_hbm.at[pl.ds(base, rows_per_w)],
                                 feat_v, sem_f)
        gathers = [None] * nchunk
        for ch in range(min(2, nchunk)):
            pltpu.sync_copy(labels_hbm.at[pl.ds(base + ch * chunk, chunk)],
                            idxs[ch])
            gathers[ch] = pltpu.async_copy(centers_hbm.at[idxs[ch]],
                                           cents[ch], sems[ch])
        fcopy.wait()

        accs = tuple(jnp.zeros((_LANES,), jnp.float32)
                     for _ in range(vecs_per_row))
        for ch in range(nchunk):
            b = ch % 2
            gathers[ch].wait()
            if ch + 2 < nchunk:
                nxt = ch + 2
                pltpu.sync_copy(
                    labels_hbm.at[pl.ds(base + nxt * chunk, chunk)], idxs[b])
                gathers[nxt] = pltpu.async_copy(centers_hbm.at[idxs[b]],
                                                cents[b], sems[b])
            cent_v = cents[b]
            row_off = ch * chunk

            def row_body(r, accs, cent_v=cent_v, row_off=row_off):
                out = []
                for j in range(vecs_per_row):
                    f = feat_v[row_off + r, pl.ds(j * _LANES, _LANES)]
                    c = cent_v[r, pl.ds(j * _LANES, _LANES)]
                    d = f - c
                    out.append(accs[j] + d * d)
                return tuple(out)

            accs = lax.fori_loop(0, chunk, row_body, accs)

        total = accs[0]
        for j in range(1, vecs_per_row):
            total = total + accs[j]
        acc_v[...] = total
        pltpu.sync_copy(acc_v, out_hbm.at[pl.ds(wid * _LANES, _LANES)])

    return sc_kernel


def kernel(features, labels, centers):
    batch, feat_dim = features.shape
    sc = _make_sc_kernel(batch, feat_dim)
    partials = sc(features, labels.astype(jnp.int32), centers)
    return jnp.sum(partials) / jnp.float32(batch)
